# baseline (speedup 1.0000x reference)
import functools

import jax
import jax.numpy as jnp
from jax import lax
from jax.experimental import pallas as pl
from jax.experimental.pallas import tpu as pltpu
from jax.experimental.pallas import tpu_sc as plsc

NC = 2
NS = 16
NW = NC * NS
G = 1000
F = 16


def _mesh():
    return plsc.VectorSubcoreMesh(core_axis_name="c", subcore_axis_name="s")


def _make_edge_agg(N, E):
    EW = E // NW
    K = 400
    NB = 4
    NCH = EW // K
    NG = (NCH + NB - 1) // NB
    ZR = K
    NZC = N // ZR

    @functools.partial(
        pl.kernel,
        out_type=jax.ShapeDtypeStruct((NC, N, F), jnp.float32),
        mesh=_mesh(),
        compiler_params=pltpu.CompilerParams(use_tc_tiling_on_sc=False),
        scratch_types=[
            [pltpu.VMEM((K,), jnp.int32)] * NB,
            [pltpu.VMEM((K,), jnp.int32)] * NB,
            [pltpu.VMEM((K, F), jnp.float32)] * NB,
            pltpu.VMEM_SHARED((N, F), jnp.float32),
            [pltpu.SemaphoreType.DMA] * NB,
            [pltpu.SemaphoreType.DMA] * NB,
            [pltpu.SemaphoreType.DMA] * NB,
        ],
    )
    def edge_agg(h_hbm, edge_hbm, out_hbm, srcbs, dstbs, rowsbs,
                 agg_sh, semi, semg, sems):
        c = lax.axis_index("c")
        s = lax.axis_index("s")
        wid = c * NS + s

        def start_idx(i, b):
            base = pl.multiple_of(wid * EW + i * K, 8)
            pltpu.async_copy(edge_hbm.at[0, pl.ds(base, K)], srcbs[b], semi[b])
            pltpu.async_copy(edge_hbm.at[1, pl.ds(base, K)], dstbs[b], semi[b])

        def wait_idx(b):
            pltpu.make_async_copy(edge_hbm.at[0, pl.ds(0, K)], srcbs[b],
                                  semi[b]).wait()
            pltpu.make_async_copy(edge_hbm.at[0, pl.ds(0, K)], dstbs[b],
                                  semi[b]).wait()

        def start_gather(b):
            pltpu.async_copy(h_hbm.at[srcbs[b]], rowsbs[b], semg[b])

        def wait_gather(b):
            pltpu.make_async_copy(h_hbm.at[srcbs[b]], rowsbs[b],
                                  semg[b]).wait()

        def start_scatter(b):
            pltpu.async_copy(rowsbs[b], agg_sh.at[dstbs[b]], sems[b],
                             add=True)

        def wait_scatter(b):
            pltpu.make_async_copy(rowsbs[b], agg_sh.at[dstbs[b]],
                                  sems[b]).wait()

        for b in range(NB):
            start_idx(b, b)

        def zb(i, carry):
            rowsbs[0][i] = jnp.zeros((F,), jnp.float32)
            return carry

        lax.fori_loop(0, K, zb, 0)

        def zsp(j, carry):
            ci = s + j * NS

            @pl.when(ci < NZC)
            def _():
                base = pl.multiple_of(ci * ZR, 8)

                @pl.when(c == 0)
                def _():
                    pltpu.sync_copy(h_hbm.at[pl.ds(base, ZR)],
                                    agg_sh.at[pl.ds(base, ZR)])

                @pl.when(c != 0)
                def _():
                    pltpu.sync_copy(rowsbs[0], agg_sh.at[pl.ds(base, ZR)])

            return carry

        lax.fori_loop(0, (NZC + NS - 1) // NS, zsp, 0)
        plsc.subcore_barrier()

        def group(j, carry):

            for b in range(NB):
                i = j * NB + b

                @pl.when(i < NCH)
                def _(b=b):
                    wait_idx(b)
                    start_gather(b)

            for b in range(NB):
                i = j * NB + b

                @pl.when(i < NCH)
                def _(b=b):
                    wait_gather(b)
                    start_scatter(b)

            for b in range(NB):
                i2 = (j + 1) * NB + b

                @pl.when(i2 < NCH)
                def _(b=b, i2=i2):
                    wait_scatter(b)
                    start_idx(i2, b)

            return carry

        lax.fori_loop(0, NG, group, 0)
        for b in range(NB):
            wait_scatter(b)
        plsc.subcore_barrier()

        def cpo(j, carry):
            ci = s + j * NS

            @pl.when(ci < NZC)
            def _():
                base = pl.multiple_of(ci * ZR, 8)
                pltpu.sync_copy(agg_sh.at[pl.ds(base, ZR)],
                                out_hbm.at[c, pl.ds(base, ZR)])

            return carry

        lax.fori_loop(0, (NZC + NS - 1) // NS, cpo, 0)

    return edge_agg


def _make_pool(N):
    CP = 1000
    NCHT = N // CP
    PERW = (NCHT + NW - 1) // NW
    GR = 200
    NGT = G // GR

    @functools.partial(
        pl.kernel,
        out_type=jax.ShapeDtypeStruct((NC, G, F), jnp.float32),
        mesh=_mesh(),
        compiler_params=pltpu.CompilerParams(use_tc_tiling_on_sc=False),
        scratch_types=[
            pltpu.VMEM((CP,), jnp.int32),
            pltpu.VMEM((CP, F), jnp.float32),
            pltpu.VMEM((GR, F), jnp.float32),
            pltpu.VMEM_SHARED((G, F), jnp.float32),
        ],
    )
    def pool(h_hbm, batch_hbm, out_hbm, bb, rowsb, zbuf, pool_sh):
        c = lax.axis_index("c")
        s = lax.axis_index("s")
        wid = c * NS + s

        def zb(i, carry):
            zbuf[i] = jnp.zeros((F,), jnp.float32)
            return carry

        lax.fori_loop(0, GR, zb, 0)

        @pl.when(s < NGT)
        def _():
            pltpu.sync_copy(zbuf, pool_sh.at[pl.ds(s * GR, GR)])

        plsc.subcore_barrier()

        def chunk(i, carry):
            ci = wid + i * NW

            @pl.when(ci < NCHT)
            def _():
                base = pl.multiple_of(ci * CP, 8)
                pltpu.sync_copy(batch_hbm.at[pl.ds(base, CP)], bb)
                pltpu.sync_copy(h_hbm.at[pl.ds(base, CP)], rowsb)
                pltpu.sync_copy(rowsb, pool_sh.at[bb], add=True)

            return carry

        lax.fori_loop(0, PERW, chunk, 0)
        plsc.subcore_barrier()

        @pl.when(s < NGT)
        def _():
            base = pl.multiple_of(s * GR, 8)
            pltpu.sync_copy(pool_sh.at[pl.ds(base, GR)],
                            out_hbm.at[c, pl.ds(base, GR)])

    return pool


def _make_mlp(N):
    R = N // 8
    NBLK = 10
    B = R // NBLK

    def body(a_ref, w1_ref, c1_ref, w2_ref, b2_ref, o_ref):
        z = a_ref[0, 0] + a_ref[1, 0]
        y = jnp.dot(z, w1_ref[...], preferred_element_type=jnp.float32,
                    precision=lax.Precision.HIGHEST)
        y = jnp.maximum(y + c1_ref[...], 0.0)
        o = jnp.dot(y, w2_ref[...], preferred_element_type=jnp.float32,
                    precision=lax.Precision.HIGHEST)
        o_ref[0] = jnp.maximum(o + b2_ref[...], 0.0)

    call = pl.pallas_call(
        body,
        grid=(NBLK,),
        in_specs=[
            pl.BlockSpec((NC, 1, B, 128), lambda i: (0, i, 0, 0)),
            pl.BlockSpec((128, 128), lambda i: (0, 0)),
            pl.BlockSpec((1, 128), lambda i: (0, 0)),
            pl.BlockSpec((128, 128), lambda i: (0, 0)),
            pl.BlockSpec((1, 128), lambda i: (0, 0)),
        ],
        out_specs=pl.BlockSpec((1, B, 128), lambda i: (i, 0, 0)),
        out_shape=jax.ShapeDtypeStruct((NBLK, B, 128), jnp.float32),
    )

    def mlp(agg128, w1bd, c1bd, w2bd, b2bd):
        ab = jnp.reshape(agg128, (NC, NBLK, B, 128))
        out = call(ab, w1bd, c1bd, w2bd, b2bd)
        return jnp.reshape(out, (R, 128))

    return mlp


def _make_head(num_layers):
    FC = num_layers * F

    def body(p_ref, w1_ref, b1_ref, w2_ref, b2_ref, o_ref):
        ps = p_ref[...]
        cols = [ps[l, 0] + ps[l, 1] for l in range(num_layers)]
        hc = jnp.concatenate(cols, axis=1)
        y = jnp.dot(hc, w1_ref[...], preferred_element_type=jnp.float32,
                    precision=lax.Precision.HIGHEST)
        y = jnp.maximum(y + b1_ref[...], 0.0)
        z = jnp.dot(y, w2_ref[...], preferred_element_type=jnp.float32,
                    precision=lax.Precision.HIGHEST)
        z = z + b2_ref[...]
        m = jnp.max(z, axis=1, keepdims=True)
        lse = m + jnp.log(jnp.sum(jnp.exp(z - m), axis=1, keepdims=True))
        o_ref[...] = z - lse

    return pl.pallas_call(
        body,
        in_specs=[
            pl.BlockSpec((num_layers, NC, G, F), lambda: (0, 0, 0, 0)),
            pl.BlockSpec((FC, FC), lambda: (0, 0)),
            pl.BlockSpec((1, FC), lambda: (0, 0)),
            pl.BlockSpec((FC, 2), lambda: (0, 0)),
            pl.BlockSpec((1, 2), lambda: (0, 0)),
        ],
        out_specs=pl.BlockSpec((G, 2), lambda: (0, 0)),
        out_shape=jax.ShapeDtypeStruct((G, 2), jnp.float32),
    )


def kernel(x, edge_index, batch, params):
    N = x.shape[0]
    E = edge_index.shape[1]
    layers = params['layers']
    nl = len(layers)

    edge_agg = _make_edge_agg(N, E)
    mlp = _make_mlp(N)
    pool = _make_pool(N)
    head = _make_head(nl)

    eye8 = jnp.eye(8, dtype=jnp.float32)
    h = x
    pools = []
    for p in layers:
        sc = p['g'] * lax.rsqrt(p['v'] + 1e-05)
        w1 = p['W1'] * sc[None, :]
        c1 = (p['b1'] - p['m']) * sc + p['be']
        din, dh = w1.shape
        w1p = jnp.zeros((F, F), jnp.float32).at[:din, :dh].set(w1)
        c1p = jnp.zeros((1, F), jnp.float32).at[0, :dh].set(c1)
        w2p = jnp.zeros((F, F), jnp.float32).at[:dh, :dh].set(p['W2'])
        b2p = jnp.zeros((1, F), jnp.float32).at[0, :dh].set(p['b2'])
        w1bd = jnp.kron(eye8, w1p)
        w2bd = jnp.kron(eye8, w2p)
        c1bd = jnp.tile(c1p, (1, 8))
        b2bd = jnp.tile(b2p, (1, 8))

        agg = edge_agg(h, edge_index)
        agg128 = jnp.reshape(agg, (NC, N // 8, 128))
        h128 = mlp(agg128, w1bd, c1bd, w2bd, b2bd)
        h = jnp.reshape(h128, (N, F))
        pools.append(pool(h, batch))

    pstack = jnp.stack(pools)

    dh = layers[0]['W2'].shape[0]
    FC = nl * F

    def pad_rows(w):
        w3 = jnp.reshape(w, (nl, dh, w.shape[1]))
        w3 = jnp.pad(w3, ((0, 0), (0, F - dh), (0, 0)))
        return jnp.reshape(w3, (nl * F, w.shape[1]))

    def pad_cols(w):
        w3 = jnp.reshape(w, (w.shape[0], nl, dh))
        w3 = jnp.pad(w3, ((0, 0), (0, 0), (0, F - dh)))
        return jnp.reshape(w3, (w.shape[0], nl * F))

    w1h = pad_cols(pad_rows(params['lin1_W']))
    b1h = pad_cols(params['lin1_b'].reshape(1, nl * dh))
    w2h = pad_rows(params['lin2_W'])
    b2h = params['lin2_b'].reshape(1, 2)

    return head(pstack, w1h, b1h, w2h, b2h)


# speedup vs baseline: 1.1311x; 1.1311x over previous
import functools

import jax
import jax.numpy as jnp
from jax import lax
from jax.experimental import pallas as pl
from jax.experimental.pallas import tpu as pltpu
from jax.experimental.pallas import tpu_sc as plsc

NC = 2
NS = 16
NW = NC * NS
G = 1000
F = 16


def _mesh():
    return plsc.VectorSubcoreMesh(core_axis_name="c", subcore_axis_name="s")


def _make_edge_agg(N, E):
    EW = E // NW
    K = 400
    NB = 4
    NCH = EW // K
    NG = (NCH + NB - 1) // NB
    ZR = K
    NZC = N // ZR

    @functools.partial(
        pl.kernel,
        out_type=jax.ShapeDtypeStruct((NC, N, F), jnp.float32),
        mesh=_mesh(),
        compiler_params=pltpu.CompilerParams(use_tc_tiling_on_sc=False),
        scratch_types=[
            [pltpu.VMEM((K,), jnp.int32)] * NB,
            [pltpu.VMEM((K,), jnp.int32)] * NB,
            [pltpu.VMEM((K, F), jnp.float32)] * NB,
            pltpu.VMEM_SHARED((N, F), jnp.float32),
            [pltpu.SemaphoreType.DMA] * NB,
            [pltpu.SemaphoreType.DMA] * NB,
            [pltpu.SemaphoreType.DMA] * NB,
            pltpu.SemaphoreType.DMA,
        ],
    )
    def edge_agg(h_hbm, edge_hbm, out_hbm, srcbs, dstbs, rowsbs,
                 agg_sh, semi, semg, sems, semz):
        c = lax.axis_index("c")
        s = lax.axis_index("s")
        wid = c * NS + s

        def start_idx(i, b):
            base = pl.multiple_of(wid * EW + i * K, 8)
            pltpu.async_copy(edge_hbm.at[0, pl.ds(base, K)], srcbs[b], semi[b])
            pltpu.async_copy(edge_hbm.at[1, pl.ds(base, K)], dstbs[b], semi[b])

        def wait_idx(b):
            pltpu.make_async_copy(edge_hbm.at[0, pl.ds(0, K)], srcbs[b],
                                  semi[b]).wait()
            pltpu.make_async_copy(edge_hbm.at[0, pl.ds(0, K)], dstbs[b],
                                  semi[b]).wait()

        def start_gather(b):
            pltpu.async_copy(h_hbm.at[srcbs[b]], rowsbs[b], semg[b])

        def wait_gather(b):
            pltpu.make_async_copy(h_hbm.at[srcbs[b]], rowsbs[b],
                                  semg[b]).wait()

        def start_scatter(b):
            pltpu.async_copy(rowsbs[b], agg_sh.at[dstbs[b]], sems[b],
                             add=True)

        def wait_scatter(b):
            pltpu.make_async_copy(rowsbs[b], agg_sh.at[dstbs[b]],
                                  sems[b]).wait()

        for b in range(NB):
            start_idx(b, b)

        def zb(i, carry):
            rowsbs[0][i] = jnp.zeros((F,), jnp.float32)
            return carry

        lax.fori_loop(0, K, zb, 0)

        def zsp(j, carry):
            ci = s + j * NS

            @pl.when(ci < NZC)
            def _():
                base = pl.multiple_of(ci * ZR, 8)
                pltpu.sync_copy(rowsbs[0], agg_sh.at[pl.ds(base, ZR)])

            return carry

        lax.fori_loop(0, (NZC + NS - 1) // NS, zsp, 0)
        plsc.subcore_barrier()

        def group(j, carry):

            for b in range(NB):
                i = j * NB + b

                @pl.when(i < NCH)
                def _(b=b):
                    wait_idx(b)
                    start_gather(b)

            for b in range(NB):
                i = j * NB + b

                @pl.when(i < NCH)
                def _(b=b):
                    wait_gather(b)
                    start_scatter(b)

            for b in range(NB):
                i2 = (j + 1) * NB + b

                @pl.when(i2 < NCH)
                def _(b=b, i2=i2):
                    wait_scatter(b)
                    start_idx(i2, b)

            return carry

        lax.fori_loop(0, NG, group, 0)
        for b in range(NB):
            wait_scatter(b)
        plsc.subcore_barrier()

        def cpo(j, carry):
            ci = s + j * NS

            @pl.when(ci < NZC)
            def _():
                base = pl.multiple_of(ci * ZR, 8)
                pltpu.sync_copy(agg_sh.at[pl.ds(base, ZR)],
                                out_hbm.at[c, pl.ds(base, ZR)])

            return carry

        lax.fori_loop(0, (NZC + NS - 1) // NS, cpo, 0)

    return edge_agg


def _make_pool(N):
    CP = 1000
    NCHT = N // CP
    PERW = (NCHT + NW - 1) // NW
    GR = 200
    NGT = G // GR

    @functools.partial(
        pl.kernel,
        out_type=jax.ShapeDtypeStruct((NC, G, F), jnp.float32),
        mesh=_mesh(),
        compiler_params=pltpu.CompilerParams(use_tc_tiling_on_sc=False),
        scratch_types=[
            pltpu.VMEM((CP,), jnp.int32),
            pltpu.VMEM((CP, F), jnp.float32),
            pltpu.VMEM((GR, F), jnp.float32),
            pltpu.VMEM_SHARED((G, F), jnp.float32),
        ],
    )
    def pool(h_hbm, batch_hbm, out_hbm, bb, rowsb, zbuf, pool_sh):
        c = lax.axis_index("c")
        s = lax.axis_index("s")
        wid = c * NS + s

        def zb(i, carry):
            zbuf[i] = jnp.zeros((F,), jnp.float32)
            return carry

        lax.fori_loop(0, GR, zb, 0)

        @pl.when(s < NGT)
        def _():
            pltpu.sync_copy(zbuf, pool_sh.at[pl.ds(s * GR, GR)])

        plsc.subcore_barrier()

        def chunk(i, carry):
            ci = wid + i * NW

            @pl.when(ci < NCHT)
            def _():
                base = pl.multiple_of(ci * CP, 8)
                pltpu.sync_copy(batch_hbm.at[pl.ds(base, CP)], bb)
                pltpu.sync_copy(h_hbm.at[pl.ds(base, CP)], rowsb)
                pltpu.sync_copy(rowsb, pool_sh.at[bb], add=True)

            return carry

        lax.fori_loop(0, PERW, chunk, 0)
        plsc.subcore_barrier()

        @pl.when(s < NGT)
        def _():
            base = pl.multiple_of(s * GR, 8)
            pltpu.sync_copy(pool_sh.at[pl.ds(base, GR)],
                            out_hbm.at[c, pl.ds(base, GR)])

    return pool


def _make_mlp(N):
    R = N // 8
    NBLK = 10
    B = R // NBLK

    def body(h_ref, a_ref, w1_ref, b1_ref, g_ref, m_ref, sq_ref, be_ref,
             w2_ref, b2_ref, o_ref):
        z = h_ref[0] + (a_ref[0, 0] + a_ref[1, 0])
        y = jnp.dot(z, w1_ref[...], preferred_element_type=jnp.float32)
        y = y + b1_ref[...]
        y = g_ref[...] * (y - m_ref[...]) / sq_ref[...] + be_ref[...]
        y = jnp.maximum(y, 0.0)
        o = jnp.dot(y, w2_ref[...], preferred_element_type=jnp.float32)
        o_ref[0] = jnp.maximum(o + b2_ref[...], 0.0)

    call = pl.pallas_call(
        body,
        grid=(NBLK,),
        in_specs=[
            pl.BlockSpec((1, B, 128), lambda i: (i, 0, 0)),
            pl.BlockSpec((NC, 1, B, 128), lambda i: (0, i, 0, 0)),
            pl.BlockSpec((128, 128), lambda i: (0, 0)),
            pl.BlockSpec((1, 128), lambda i: (0, 0)),
            pl.BlockSpec((1, 128), lambda i: (0, 0)),
            pl.BlockSpec((1, 128), lambda i: (0, 0)),
            pl.BlockSpec((1, 128), lambda i: (0, 0)),
            pl.BlockSpec((1, 128), lambda i: (0, 0)),
            pl.BlockSpec((128, 128), lambda i: (0, 0)),
            pl.BlockSpec((1, 128), lambda i: (0, 0)),
        ],
        out_specs=pl.BlockSpec((1, B, 128), lambda i: (i, 0, 0)),
        out_shape=jax.ShapeDtypeStruct((NBLK, B, 128), jnp.float32),
    )

    def mlp(h128, agg128, *ws):
        hb = jnp.reshape(h128, (NBLK, B, 128))
        ab = jnp.reshape(agg128, (NC, NBLK, B, 128))
        out = call(hb, ab, *ws)
        return jnp.reshape(out, (R, 128))

    return mlp


def _make_head(num_layers):
    FC = num_layers * F

    def body(p_ref, w1_ref, b1_ref, w2_ref, b2_ref, o_ref):
        ps = p_ref[...]
        cols = [ps[l, 0] + ps[l, 1] for l in range(num_layers)]
        hc = jnp.concatenate(cols, axis=1)
        y = jnp.dot(hc, w1_ref[...], preferred_element_type=jnp.float32)
        y = jnp.maximum(y + b1_ref[...], 0.0)
        z = jnp.dot(y, w2_ref[...], preferred_element_type=jnp.float32)
        z = z + b2_ref[...]
        m = jnp.max(z, axis=1, keepdims=True)
        lse = m + jnp.log(jnp.sum(jnp.exp(z - m), axis=1, keepdims=True))
        o_ref[...] = z - lse

    return pl.pallas_call(
        body,
        in_specs=[
            pl.BlockSpec((num_layers, NC, G, F), lambda: (0, 0, 0, 0)),
            pl.BlockSpec((FC, FC), lambda: (0, 0)),
            pl.BlockSpec((1, FC), lambda: (0, 0)),
            pl.BlockSpec((FC, 2), lambda: (0, 0)),
            pl.BlockSpec((1, 2), lambda: (0, 0)),
        ],
        out_specs=pl.BlockSpec((G, 2), lambda: (0, 0)),
        out_shape=jax.ShapeDtypeStruct((G, 2), jnp.float32),
    )


def kernel(x, edge_index, batch, params):
    N = x.shape[0]
    E = edge_index.shape[1]
    layers = params['layers']
    nl = len(layers)

    edge_agg = _make_edge_agg(N, E)
    mlp = _make_mlp(N)
    pool = _make_pool(N)
    head = _make_head(nl)

    eye8 = jnp.eye(8, dtype=jnp.float32)
    h = x
    h128 = jnp.reshape(x, (N // 8, 128))
    pools = []
    def pad16(v, fill, dh):
        return jnp.full((1, F), fill, jnp.float32).at[0, :dh].set(v)

    for p in layers:
        din, dh = p['W1'].shape
        w1p = jnp.zeros((F, F), jnp.float32).at[:din, :dh].set(p['W1'])
        w2p = jnp.zeros((F, F), jnp.float32).at[:dh, :dh].set(p['W2'])
        w1bd = jnp.kron(eye8, w1p)
        w2bd = jnp.kron(eye8, w2p)
        t8 = lambda a: jnp.tile(a, (1, 8))
        b1bd = t8(pad16(p['b1'], 0.0, dh))
        gbd = t8(pad16(p['g'], 1.0, dh))
        mbd = t8(pad16(p['m'], 0.0, dh))
        sqbd = t8(pad16(jnp.sqrt(p['v'] + 1e-05), 1.0, dh))
        bebd = t8(pad16(p['be'], 0.0, dh))
        b2bd = t8(pad16(p['b2'], 0.0, dh))

        agg = edge_agg(h, edge_index)
        agg128 = jnp.reshape(agg, (NC, N // 8, 128))
        h128 = mlp(h128, agg128, w1bd, b1bd, gbd, mbd, sqbd, bebd,
                   w2bd, b2bd)
        h = jnp.reshape(h128, (N, F))
        pools.append(pool(h, batch))

    pstack = jnp.stack(pools)

    dh = layers[0]['W2'].shape[0]
    FC = nl * F

    def pad_rows(w):
        w3 = jnp.reshape(w, (nl, dh, w.shape[1]))
        w3 = jnp.pad(w3, ((0, 0), (0, F - dh), (0, 0)))
        return jnp.reshape(w3, (nl * F, w.shape[1]))

    def pad_cols(w):
        w3 = jnp.reshape(w, (w.shape[0], nl, dh))
        w3 = jnp.pad(w3, ((0, 0), (0, 0), (0, F - dh)))
        return jnp.reshape(w3, (w.shape[0], nl * F))

    w1h = pad_cols(pad_rows(params['lin1_W']))
    b1h = pad_cols(params['lin1_b'].reshape(1, nl * dh))
    w2h = pad_rows(params['lin2_W'])
    b2h = params['lin2_b'].reshape(1, 2)

    return head(pstack, w1h, b1h, w2h, b2h)
